# baseline (speedup 1.0000x reference)
import functools

import jax
import jax.numpy as jnp
from jax import lax
from jax.experimental import pallas as pl
from jax.experimental.pallas import tpu as pltpu
from jax.experimental.pallas import tpu_sc as plsc

_N = 5000
_B = 512
_NPAD = 5120
_NB = _NPAD // _B
_D = 16


def _make_sc_gather():
    info = plsc.get_sparse_core_info()
    nw = info.num_cores * info.num_subcores
    b_per_w = _NPAD // nw
    mesh = plsc.VectorSubcoreMesh(core_axis_name="c", subcore_axis_name="s")

    @functools.partial(
        pl.kernel,
        mesh=mesh,
        compiler_params=pltpu.CompilerParams(use_tc_tiling_on_sc=False),
        out_type=jax.ShapeDtypeStruct((_NPAD, _D), jnp.float32),
        scratch_types=[
            pltpu.VMEM((b_per_w,), jnp.int32),
            pltpu.VMEM((b_per_w, _D), jnp.float32),
            pltpu.SemaphoreType.DMA,
        ],
    )
    def sc_gather(table_hbm, idx_hbm, out_hbm, idx_v, rows_v, sem):
        wid = lax.axis_index("s") * info.num_cores + lax.axis_index("c")
        base = wid * b_per_w
        pltpu.sync_copy(idx_hbm.at[pl.ds(base, b_per_w)], idx_v)
        pltpu.async_copy(table_hbm.at[idx_v], rows_v, sem).wait()
        pltpu.sync_copy(rows_v, out_hbm.at[pl.ds(base, b_per_w)])

    return sc_gather


_sc_gather = _make_sc_gather()


def _weight_kernel(scores_ref, unc_ref, w_ref):
    m = jnp.mean(unc_ref[...], axis=0, keepdims=True)
    w_ref[...] = scores_ref[...] * (1.0 - jax.nn.sigmoid(m))


def _nms_kernel(
    xs0_ref, ys0_ref, xs1_ref, ys1_ref, a3_ref, bt_ref, keep_ref, tile_ref, supp_ref
):
    tri = (
        jax.lax.broadcasted_iota(jnp.int32, (_B, _B), 1)
        > jax.lax.broadcasted_iota(jnp.int32, (_B, _B), 0)
    )
    supp_ref[...] = jnp.zeros((1, _NPAD), jnp.float32)

    def block_body(bi, carry):
        r0 = bi * _B
        rx0 = xs0_ref[pl.ds(r0, _B), :]
        ry0 = ys0_ref[pl.ds(r0, _B), :]
        rx1 = xs1_ref[pl.ds(r0, _B), :]
        ry1 = ys1_ref[pl.ds(r0, _B), :]
        ra3 = a3_ref[pl.ds(r0, _B), :]

        def iou_bin(c0):
            cx0 = bt_ref[0:1, pl.ds(c0, _B)]
            cy0 = bt_ref[1:2, pl.ds(c0, _B)]
            cx1 = bt_ref[2:3, pl.ds(c0, _B)]
            cy1 = bt_ref[3:4, pl.ds(c0, _B)]
            ca3 = bt_ref[4:5, pl.ds(c0, _B)]
            w = jnp.maximum(jnp.minimum(rx1, cx1) - jnp.maximum(rx0, cx0), 0.0)
            h = jnp.maximum(jnp.minimum(ry1, cy1) - jnp.maximum(ry0, cy0), 0.0)
            return w * h > (ra3 + ca3)

        tile_ref[...] = jnp.where(
            iou_bin(r0) & tri, 1.0, 0.0
        ).astype(jnp.bfloat16)
        active = 1.0 - supp_ref[:, pl.ds(r0, _B)]

        def w_cond(c):
            return c[1]

        def w_body(c):
            k, _ = c
            t = jnp.dot(
                k.astype(jnp.bfloat16),
                tile_ref[...],
                preferred_element_type=jnp.float32,
            )
            k_new = active * jnp.where(t < 0.5, 1.0, 0.0)
            return k_new, jnp.any(k_new != k)

        k, _ = jax.lax.while_loop(w_cond, w_body, (active, jnp.bool_(True)))
        supp_ref[:, pl.ds(r0, _B)] = 1.0 - k
        kb = k.astype(jnp.bfloat16)

        def col_body(j, carry2):
            @pl.when(j > bi)
            def _():
                c0 = j * _B
                tile = jnp.where(iou_bin(c0), 1.0, 0.0).astype(jnp.bfloat16)
                t = jnp.dot(kb, tile, preferred_element_type=jnp.float32)
                supp_ref[:, pl.ds(c0, _B)] = jnp.maximum(
                    supp_ref[:, pl.ds(c0, _B)],
                    jnp.where(t > 0.5, 1.0, 0.0),
                )

            return carry2

        jax.lax.fori_loop(0, _NB, col_body, carry, unroll=5)
        return carry

    jax.lax.fori_loop(0, _NB, block_body, 0)
    keep_ref[...] = 1.0 - supp_ref[...]


def _assemble_kernel(boxes_ref, scores_ref, unc_ref, w_ref, cont_ref, keep_ref, out_ref):
    k = keep_ref[...]
    out_ref[:, 0:4] = boxes_ref[...] * k
    out_ref[:, 4:5] = scores_ref[...] * k
    out_ref[:, 5:9] = unc_ref[...] * k
    out_ref[:, 9:10] = w_ref[...] * k
    out_ref[:, 10:74] = cont_ref[...] * k


def kernel(contours, scores, boxes, uncertainties):
    scores_row = scores.reshape(1, _N)
    unc_t = uncertainties.T
    w_row = pl.pallas_call(
        _weight_kernel,
        out_shape=jax.ShapeDtypeStruct((1, _N), jnp.float32),
    )(scores_row, unc_t)
    wflat = w_row.reshape(_N)
    order = jnp.argsort(-wflat)
    area3 = (boxes[:, 2] - boxes[:, 0]) * (boxes[:, 3] - boxes[:, 1]) / 3.0
    table = (
        jnp.zeros((_NPAD, _D), jnp.float32)
        .at[:_N, :4].set(boxes)
        .at[:_N, 4].set(area3)
    )
    order_pad = jnp.concatenate(
        [order.astype(jnp.int32), jnp.full((_NPAD - _N,), _N, jnp.int32)]
    )
    sorted_table = _sc_gather(table, order_pad)
    cols_pad = sorted_table[:, :5]
    boxes_t = jnp.zeros((8, _NPAD), jnp.float32).at[:5].set(cols_pad.T)
    keep_row = pl.pallas_call(
        _nms_kernel,
        out_shape=jax.ShapeDtypeStruct((1, _NPAD), jnp.float32),
        scratch_shapes=[
            pltpu.VMEM((_B, _B), jnp.bfloat16),
            pltpu.VMEM((1, _NPAD), jnp.float32),
        ],
    )(
        cols_pad[:, 0:1],
        cols_pad[:, 1:2],
        cols_pad[:, 2:3],
        cols_pad[:, 3:4],
        cols_pad[:, 4:5],
        boxes_t,
    )
    _, keep = jax.lax.sort(
        (order.astype(jnp.int32), keep_row[0, :_N]), num_keys=1
    )
    out = pl.pallas_call(
        _assemble_kernel,
        out_shape=jax.ShapeDtypeStruct((_N, 74), jnp.float32),
    )(
        boxes,
        scores[:, None],
        uncertainties,
        wflat[:, None],
        contours.reshape(_N, -1),
        keep[:, None],
    )
    return out


# speedup vs baseline: 1.3073x; 1.3073x over previous
import functools

import jax
import jax.numpy as jnp
from jax import lax
from jax.experimental import pallas as pl
from jax.experimental.pallas import tpu as pltpu
from jax.experimental.pallas import tpu_sc as plsc

_N = 5000
_B = 512
_NPAD = 5120
_NB = _NPAD // _B
_D = 16


def _make_sc_gather():
    info = plsc.get_sparse_core_info()
    nw = info.num_cores * info.num_subcores
    b_per_w = _NPAD // nw
    mesh = plsc.VectorSubcoreMesh(core_axis_name="c", subcore_axis_name="s")

    @functools.partial(
        pl.kernel,
        mesh=mesh,
        compiler_params=pltpu.CompilerParams(use_tc_tiling_on_sc=False),
        out_type=jax.ShapeDtypeStruct((_NPAD, _D), jnp.float32),
        scratch_types=[
            pltpu.VMEM((b_per_w,), jnp.int32),
            pltpu.VMEM((b_per_w, _D), jnp.float32),
            pltpu.SemaphoreType.DMA,
        ],
    )
    def sc_gather(table_hbm, idx_hbm, out_hbm, idx_v, rows_v, sem):
        wid = lax.axis_index("s") * info.num_cores + lax.axis_index("c")
        base = wid * b_per_w
        pltpu.sync_copy(idx_hbm.at[pl.ds(base, b_per_w)], idx_v)
        pltpu.async_copy(table_hbm.at[idx_v], rows_v, sem).wait()
        pltpu.sync_copy(rows_v, out_hbm.at[pl.ds(base, b_per_w)])

    return sc_gather


_sc_gather = _make_sc_gather()


def _prep_kernel(scores_ref, unc_ref, boxes_ref, wcol_ref, negw_ref, table_ref):
    u = unc_ref[...]
    m = (u[:, 0:1] + u[:, 1:2] + u[:, 2:3] + u[:, 3:4]) * 0.25
    w = scores_ref[...] * (1.0 - jax.nn.sigmoid(m))
    wcol_ref[...] = w
    negw_ref[...] = -w
    b = boxes_ref[...]
    a3 = (b[:, 2:3] - b[:, 0:1]) * (b[:, 3:4] - b[:, 1:2]) / 3.0
    table_ref[...] = jnp.zeros((_NPAD, _D), jnp.float32)
    table_ref[pl.ds(0, _N), 0:4] = b
    table_ref[pl.ds(0, _N), 4:5] = a3


def _nms_kernel(st_ref, keep_ref, bt_ref, tile_ref, supp_ref):
    tri = (
        jax.lax.broadcasted_iota(jnp.int32, (_B, _B), 1)
        > jax.lax.broadcasted_iota(jnp.int32, (_B, _B), 0)
    )
    supp_ref[...] = jnp.zeros((1, _NPAD), jnp.float32)

    def bt_body(j, carry):
        c0 = j * _B
        bt_ref[:, pl.ds(c0, _B)] = st_ref[pl.ds(c0, _B), :].T
        return carry

    jax.lax.fori_loop(0, _NB, bt_body, 0)

    def block_body(bi, carry):
        r0 = bi * _B
        rx0 = st_ref[pl.ds(r0, _B), 0:1]
        ry0 = st_ref[pl.ds(r0, _B), 1:2]
        rx1 = st_ref[pl.ds(r0, _B), 2:3]
        ry1 = st_ref[pl.ds(r0, _B), 3:4]
        ra3 = st_ref[pl.ds(r0, _B), 4:5]

        def iou_bin(c0):
            cx0 = bt_ref[0:1, pl.ds(c0, _B)]
            cy0 = bt_ref[1:2, pl.ds(c0, _B)]
            cx1 = bt_ref[2:3, pl.ds(c0, _B)]
            cy1 = bt_ref[3:4, pl.ds(c0, _B)]
            ca3 = bt_ref[4:5, pl.ds(c0, _B)]
            w = jnp.maximum(jnp.minimum(rx1, cx1) - jnp.maximum(rx0, cx0), 0.0)
            h = jnp.maximum(jnp.minimum(ry1, cy1) - jnp.maximum(ry0, cy0), 0.0)
            return w * h > (ra3 + ca3)

        tile_ref[...] = jnp.where(
            iou_bin(r0) & tri, 1.0, 0.0
        ).astype(jnp.bfloat16)
        active = 1.0 - supp_ref[:, pl.ds(r0, _B)]

        def w_cond(c):
            return c[1]

        def w_body(c):
            k, _ = c
            t = jnp.dot(
                k.astype(jnp.bfloat16),
                tile_ref[...],
                preferred_element_type=jnp.float32,
            )
            k_new = active * jnp.where(t < 0.5, 1.0, 0.0)
            return k_new, jnp.any(k_new != k)

        k, _ = jax.lax.while_loop(w_cond, w_body, (active, jnp.bool_(True)))
        supp_ref[:, pl.ds(r0, _B)] = 1.0 - k
        kb = k.astype(jnp.bfloat16)

        def col_body(j, carry2):
            @pl.when(j > bi)
            def _():
                c0 = j * _B
                tile = jnp.where(iou_bin(c0), 1.0, 0.0).astype(jnp.bfloat16)
                t = jnp.dot(kb, tile, preferred_element_type=jnp.float32)
                supp_ref[:, pl.ds(c0, _B)] = jnp.maximum(
                    supp_ref[:, pl.ds(c0, _B)],
                    jnp.where(t > 0.5, 1.0, 0.0),
                )

            return carry2

        jax.lax.fori_loop(0, _NB, col_body, carry, unroll=5)
        return carry

    jax.lax.fori_loop(0, _NB, block_body, 0)
    keep_ref[...] = 1.0 - supp_ref[...]


def _assemble_kernel(boxes_ref, scores_ref, unc_ref, w_ref, cont_ref, keep_ref, out_ref):
    k = keep_ref[...]
    out_ref[:, 0:4] = boxes_ref[...] * k
    out_ref[:, 4:5] = scores_ref[...] * k
    out_ref[:, 5:9] = unc_ref[...] * k
    out_ref[:, 9:10] = w_ref[...] * k
    out_ref[:, 10:74] = cont_ref[...] * k


def kernel(contours, scores, boxes, uncertainties):
    wcol, negw, table = pl.pallas_call(
        _prep_kernel,
        out_shape=(
            jax.ShapeDtypeStruct((_N, 1), jnp.float32),
            jax.ShapeDtypeStruct((_N, 1), jnp.float32),
            jax.ShapeDtypeStruct((_NPAD, _D), jnp.float32),
        ),
    )(scores[:, None], uncertainties, boxes)
    order = jnp.argsort(negw.reshape(_N)).astype(jnp.int32)
    order_pad = jnp.concatenate(
        [order, jnp.full((_NPAD - _N,), _N, jnp.int32)]
    )
    sorted_table = _sc_gather(table, order_pad)
    keep_row = pl.pallas_call(
        _nms_kernel,
        out_shape=jax.ShapeDtypeStruct((1, _NPAD), jnp.float32),
        scratch_shapes=[
            pltpu.VMEM((_D, _NPAD), jnp.float32),
            pltpu.VMEM((_B, _B), jnp.bfloat16),
            pltpu.VMEM((1, _NPAD), jnp.float32),
        ],
    )(sorted_table)
    _, keep = jax.lax.sort((order, keep_row[0, :_N]), num_keys=1)
    out = pl.pallas_call(
        _assemble_kernel,
        out_shape=jax.ShapeDtypeStruct((_N, 74), jnp.float32),
    )(
        boxes,
        scores[:, None],
        uncertainties,
        wcol,
        contours.reshape(_N, -1),
        keep[:, None],
    )
    return out


# speedup vs baseline: 1.3674x; 1.0460x over previous
import functools

import jax
import jax.numpy as jnp
from jax import lax
from jax.experimental import pallas as pl
from jax.experimental.pallas import tpu as pltpu
from jax.experimental.pallas import tpu_sc as plsc

_N = 5000
_B = 512
_NPAD = 5120
_NB = _NPAD // _B
_D = 16


def _make_sc_gather():
    info = plsc.get_sparse_core_info()
    nw = info.num_cores * info.num_subcores
    b_per_w = _NPAD // nw
    mesh = plsc.VectorSubcoreMesh(core_axis_name="c", subcore_axis_name="s")

    @functools.partial(
        pl.kernel,
        mesh=mesh,
        compiler_params=pltpu.CompilerParams(use_tc_tiling_on_sc=False),
        out_type=jax.ShapeDtypeStruct((_NPAD, _D), jnp.float32),
        scratch_types=[
            pltpu.VMEM((b_per_w,), jnp.int32),
            pltpu.VMEM((b_per_w, _D), jnp.float32),
            pltpu.SemaphoreType.DMA,
        ],
    )
    def sc_gather(table_hbm, idx_hbm, out_hbm, idx_v, rows_v, sem):
        wid = lax.axis_index("s") * info.num_cores + lax.axis_index("c")
        base = wid * b_per_w
        pltpu.sync_copy(idx_hbm.at[pl.ds(base, b_per_w)], idx_v)
        pltpu.async_copy(table_hbm.at[idx_v], rows_v, sem).wait()
        pltpu.sync_copy(rows_v, out_hbm.at[pl.ds(base, b_per_w)])

    return sc_gather


_sc_gather = _make_sc_gather()


def _prep_kernel(scores_ref, unc_ref, boxes_ref, wcol_ref, negw_ref, table_ref):
    u = unc_ref[...]
    m = (u[:, 0:1] + u[:, 1:2] + u[:, 2:3] + u[:, 3:4]) * 0.25
    w = scores_ref[...].T * (1.0 - jax.nn.sigmoid(m))
    wcol_ref[...] = w
    negw_ref[0:1, 0:_N] = -w.T
    negw_ref[0:1, _N:_NPAD] = jnp.full((1, _NPAD - _N), jnp.inf, jnp.float32)
    b = boxes_ref[...]
    a3 = (b[:, 2:3] - b[:, 0:1]) * (b[:, 3:4] - b[:, 1:2]) / 3.0
    table_ref[...] = jnp.zeros((_NPAD, _D), jnp.float32)
    table_ref[pl.ds(0, _N), 0:4] = b
    table_ref[pl.ds(0, _N), 4:5] = a3


def _nms_kernel(st_ref, keep_ref, bt_ref, tile_ref, supp_ref):
    tri = (
        jax.lax.broadcasted_iota(jnp.int32, (_B, _B), 1)
        > jax.lax.broadcasted_iota(jnp.int32, (_B, _B), 0)
    )
    supp_ref[...] = jnp.zeros((1, _NPAD), jnp.float32)

    def bt_body(j, carry):
        c0 = j * _B
        bt_ref[:, pl.ds(c0, _B)] = st_ref[pl.ds(c0, _B), :].T
        return carry

    jax.lax.fori_loop(0, _NB, bt_body, 0)

    def block_body(bi, carry):
        r0 = bi * _B
        rx0 = st_ref[pl.ds(r0, _B), 0:1]
        ry0 = st_ref[pl.ds(r0, _B), 1:2]
        rx1 = st_ref[pl.ds(r0, _B), 2:3]
        ry1 = st_ref[pl.ds(r0, _B), 3:4]
        ra3 = st_ref[pl.ds(r0, _B), 4:5]

        def iou_bin(c0):
            cx0 = bt_ref[0:1, pl.ds(c0, _B)]
            cy0 = bt_ref[1:2, pl.ds(c0, _B)]
            cx1 = bt_ref[2:3, pl.ds(c0, _B)]
            cy1 = bt_ref[3:4, pl.ds(c0, _B)]
            ca3 = bt_ref[4:5, pl.ds(c0, _B)]
            w = jnp.maximum(jnp.minimum(rx1, cx1) - jnp.maximum(rx0, cx0), 0.0)
            h = jnp.maximum(jnp.minimum(ry1, cy1) - jnp.maximum(ry0, cy0), 0.0)
            return w * h > (ra3 + ca3)

        tile_ref[...] = jnp.where(
            iou_bin(r0) & tri, 1.0, 0.0
        ).astype(jnp.bfloat16)
        active = 1.0 - supp_ref[:, pl.ds(r0, _B)]

        def w_cond(c):
            return c[1]

        def w_body(c):
            k, _ = c
            t = jnp.dot(
                k.astype(jnp.bfloat16),
                tile_ref[...],
                preferred_element_type=jnp.float32,
            )
            k_new = active * jnp.where(t < 0.5, 1.0, 0.0)
            return k_new, jnp.any(k_new != k)

        k, _ = jax.lax.while_loop(w_cond, w_body, (active, jnp.bool_(True)))
        supp_ref[:, pl.ds(r0, _B)] = 1.0 - k
        kb = k.astype(jnp.bfloat16)

        def col_body(j, carry2):
            @pl.when(j > bi)
            def _():
                c0 = j * _B
                tile = jnp.where(iou_bin(c0), 1.0, 0.0).astype(jnp.bfloat16)
                t = jnp.dot(kb, tile, preferred_element_type=jnp.float32)
                supp_ref[:, pl.ds(c0, _B)] = jnp.maximum(
                    supp_ref[:, pl.ds(c0, _B)],
                    jnp.where(t > 0.5, 1.0, 0.0),
                )

            return carry2

        jax.lax.fori_loop(0, _NB, col_body, carry, unroll=5)
        return carry

    jax.lax.fori_loop(0, _NB, block_body, 0)
    keep_ref[...] = 1.0 - supp_ref[...]


def _assemble_kernel(boxes_ref, scores_ref, unc_ref, w_ref, cont_ref, keep_ref, out_ref):
    k = keep_ref[0:1, 0:_N].T
    out_ref[:, 0:4] = boxes_ref[...] * k
    out_ref[:, 4:5] = scores_ref[...].T * k
    out_ref[:, 5:9] = unc_ref[...] * k
    out_ref[:, 9:10] = w_ref[...] * k
    out_ref[:, 10:74] = cont_ref[...] * k


def kernel(contours, scores, boxes, uncertainties):
    wcol, negw, table = pl.pallas_call(
        _prep_kernel,
        out_shape=(
            jax.ShapeDtypeStruct((_N, 1), jnp.float32),
            jax.ShapeDtypeStruct((1, _NPAD), jnp.float32),
            jax.ShapeDtypeStruct((_NPAD, _D), jnp.float32),
        ),
    )(scores.reshape(1, _N), uncertainties, boxes)
    order_pad = jnp.argsort(negw.reshape(_NPAD)).astype(jnp.int32)
    sorted_table = _sc_gather(table, order_pad)
    keep_row = pl.pallas_call(
        _nms_kernel,
        out_shape=jax.ShapeDtypeStruct((1, _NPAD), jnp.float32),
        scratch_shapes=[
            pltpu.VMEM((_D, _NPAD), jnp.float32),
            pltpu.VMEM((_B, _B), jnp.bfloat16),
            pltpu.VMEM((1, _NPAD), jnp.float32),
        ],
    )(sorted_table)
    _, keep_pad = jax.lax.sort((order_pad, keep_row.reshape(_NPAD)), num_keys=1)
    out = pl.pallas_call(
        _assemble_kernel,
        out_shape=jax.ShapeDtypeStruct((_N, 74), jnp.float32),
    )(
        boxes,
        scores.reshape(1, _N),
        uncertainties,
        wcol,
        contours.reshape(_N, -1),
        keep_pad.reshape(1, _NPAD),
    )
    return out


# speedup vs baseline: 1.4340x; 1.0487x over previous
import functools

import jax
import jax.numpy as jnp
from jax import lax
from jax.experimental import pallas as pl
from jax.experimental.pallas import tpu as pltpu
from jax.experimental.pallas import tpu_sc as plsc

_N = 5000
_B = 1024
_NPAD = 5120
_NB = _NPAD // _B
_D = 16


def _make_sc_gather():
    info = plsc.get_sparse_core_info()
    nw = info.num_cores * info.num_subcores
    b_per_w = _NPAD // nw
    mesh = plsc.VectorSubcoreMesh(core_axis_name="c", subcore_axis_name="s")

    @functools.partial(
        pl.kernel,
        mesh=mesh,
        compiler_params=pltpu.CompilerParams(use_tc_tiling_on_sc=False),
        out_type=jax.ShapeDtypeStruct((_NPAD, _D), jnp.float32),
        scratch_types=[
            pltpu.VMEM((b_per_w,), jnp.int32),
            pltpu.VMEM((b_per_w, _D), jnp.float32),
            pltpu.SemaphoreType.DMA,
        ],
    )
    def sc_gather(table_hbm, idx_hbm, out_hbm, idx_v, rows_v, sem):
        wid = lax.axis_index("s") * info.num_cores + lax.axis_index("c")
        base = wid * b_per_w
        pltpu.sync_copy(idx_hbm.at[pl.ds(base, b_per_w)], idx_v)
        pltpu.async_copy(table_hbm.at[idx_v], rows_v, sem).wait()
        pltpu.sync_copy(rows_v, out_hbm.at[pl.ds(base, b_per_w)])

    return sc_gather


_sc_gather = _make_sc_gather()


def _prep_kernel(scores_ref, unc_ref, boxes_ref, wcol_ref, negw_ref, table_ref):
    u = unc_ref[...]
    m = (u[:, 0:1] + u[:, 1:2] + u[:, 2:3] + u[:, 3:4]) * 0.25
    w = scores_ref[...].T * (1.0 - jax.nn.sigmoid(m))
    wcol_ref[...] = w
    negw_ref[0:1, 0:_N] = -w.T
    negw_ref[0:1, _N:_NPAD] = jnp.full((1, _NPAD - _N), jnp.inf, jnp.float32)
    b = boxes_ref[...]
    a3 = (b[:, 2:3] - b[:, 0:1]) * (b[:, 3:4] - b[:, 1:2]) / 3.0
    table_ref[...] = jnp.zeros((_NPAD, _D), jnp.float32)
    table_ref[pl.ds(0, _N), 0:4] = b
    table_ref[pl.ds(0, _N), 4:5] = a3


def _nms_kernel(st_ref, keep_ref, bt_ref, tile_ref, supp_ref):
    tri = (
        jax.lax.broadcasted_iota(jnp.int32, (_B, _B), 1)
        > jax.lax.broadcasted_iota(jnp.int32, (_B, _B), 0)
    )
    supp_ref[...] = jnp.zeros((1, _NPAD), jnp.float32)

    def bt_body(j, carry):
        c0 = j * _B
        bt_ref[:, pl.ds(c0, _B)] = st_ref[pl.ds(c0, _B), :].T
        return carry

    jax.lax.fori_loop(0, _NB, bt_body, 0)

    def block_body(bi, carry):
        r0 = bi * _B
        rx0 = st_ref[pl.ds(r0, _B), 0:1]
        ry0 = st_ref[pl.ds(r0, _B), 1:2]
        rx1 = st_ref[pl.ds(r0, _B), 2:3]
        ry1 = st_ref[pl.ds(r0, _B), 3:4]
        ra3 = st_ref[pl.ds(r0, _B), 4:5]

        def iou_bin(c0):
            cx0 = bt_ref[0:1, pl.ds(c0, _B)]
            cy0 = bt_ref[1:2, pl.ds(c0, _B)]
            cx1 = bt_ref[2:3, pl.ds(c0, _B)]
            cy1 = bt_ref[3:4, pl.ds(c0, _B)]
            ca3 = bt_ref[4:5, pl.ds(c0, _B)]
            w = jnp.maximum(jnp.minimum(rx1, cx1) - jnp.maximum(rx0, cx0), 0.0)
            h = jnp.maximum(jnp.minimum(ry1, cy1) - jnp.maximum(ry0, cy0), 0.0)
            return w * h > (ra3 + ca3)

        tile_ref[...] = jnp.where(
            iou_bin(r0) & tri, 1.0, 0.0
        ).astype(jnp.bfloat16)
        active = 1.0 - supp_ref[:, pl.ds(r0, _B)]

        def w_cond(c):
            return c[1]

        def w_body(c):
            k, _ = c
            t = jnp.dot(
                k.astype(jnp.bfloat16),
                tile_ref[...],
                preferred_element_type=jnp.float32,
            )
            k_new = active * jnp.where(t < 0.5, 1.0, 0.0)
            return k_new, jnp.any(k_new != k)

        k, _ = jax.lax.while_loop(w_cond, w_body, (active, jnp.bool_(True)))
        supp_ref[:, pl.ds(r0, _B)] = 1.0 - k
        kb = k.astype(jnp.bfloat16)

        def col_body(j, carry2):
            @pl.when(j > bi)
            def _():
                c0 = j * _B
                tile = jnp.where(iou_bin(c0), 1.0, 0.0).astype(jnp.bfloat16)
                t = jnp.dot(kb, tile, preferred_element_type=jnp.float32)
                supp_ref[:, pl.ds(c0, _B)] = jnp.maximum(
                    supp_ref[:, pl.ds(c0, _B)],
                    jnp.where(t > 0.5, 1.0, 0.0),
                )

            return carry2

        jax.lax.fori_loop(0, _NB, col_body, carry, unroll=5)
        return carry

    jax.lax.fori_loop(0, _NB, block_body, 0)
    keep_ref[...] = 1.0 - supp_ref[...]


def _assemble_kernel(boxes_ref, scores_ref, unc_ref, w_ref, cont_ref, keep_ref, out_ref):
    k = keep_ref[0:1, 0:_N].T
    out_ref[:, 0:4] = boxes_ref[...] * k
    out_ref[:, 4:5] = scores_ref[...].T * k
    out_ref[:, 5:9] = unc_ref[...] * k
    out_ref[:, 9:10] = w_ref[...] * k
    out_ref[:, 10:74] = cont_ref[...] * k


def kernel(contours, scores, boxes, uncertainties):
    wcol, negw, table = pl.pallas_call(
        _prep_kernel,
        out_shape=(
            jax.ShapeDtypeStruct((_N, 1), jnp.float32),
            jax.ShapeDtypeStruct((1, _NPAD), jnp.float32),
            jax.ShapeDtypeStruct((_NPAD, _D), jnp.float32),
        ),
    )(scores.reshape(1, _N), uncertainties, boxes)
    order_pad = jnp.argsort(negw.reshape(_NPAD)).astype(jnp.int32)
    sorted_table = _sc_gather(table, order_pad)
    keep_row = pl.pallas_call(
        _nms_kernel,
        out_shape=jax.ShapeDtypeStruct((1, _NPAD), jnp.float32),
        scratch_shapes=[
            pltpu.VMEM((_D, _NPAD), jnp.float32),
            pltpu.VMEM((_B, _B), jnp.bfloat16),
            pltpu.VMEM((1, _NPAD), jnp.float32),
        ],
    )(sorted_table)
    _, keep_pad = jax.lax.sort((order_pad, keep_row.reshape(_NPAD)), num_keys=1)
    out = pl.pallas_call(
        _assemble_kernel,
        out_shape=jax.ShapeDtypeStruct((_N, 74), jnp.float32),
    )(
        boxes,
        scores.reshape(1, _N),
        uncertainties,
        wcol,
        contours.reshape(_N, -1),
        keep_pad.reshape(1, _NPAD),
    )
    return out


# speedup vs baseline: 1.4341x; 1.0001x over previous
import functools

import jax
import jax.numpy as jnp
from jax import lax
from jax.experimental import pallas as pl
from jax.experimental.pallas import tpu as pltpu
from jax.experimental.pallas import tpu_sc as plsc

_N = 5000
_B = 1280
_NPAD = 5120
_NB = _NPAD // _B
_D = 16


def _make_sc_gather():
    info = plsc.get_sparse_core_info()
    nw = info.num_cores * info.num_subcores
    b_per_w = _NPAD // nw
    mesh = plsc.VectorSubcoreMesh(core_axis_name="c", subcore_axis_name="s")

    @functools.partial(
        pl.kernel,
        mesh=mesh,
        compiler_params=pltpu.CompilerParams(use_tc_tiling_on_sc=False),
        out_type=jax.ShapeDtypeStruct((_NPAD, _D), jnp.float32),
        scratch_types=[
            pltpu.VMEM((b_per_w,), jnp.int32),
            pltpu.VMEM((b_per_w, _D), jnp.float32),
            pltpu.SemaphoreType.DMA,
        ],
    )
    def sc_gather(table_hbm, idx_hbm, out_hbm, idx_v, rows_v, sem):
        wid = lax.axis_index("s") * info.num_cores + lax.axis_index("c")
        base = wid * b_per_w
        pltpu.sync_copy(idx_hbm.at[pl.ds(base, b_per_w)], idx_v)
        pltpu.async_copy(table_hbm.at[idx_v], rows_v, sem).wait()
        pltpu.sync_copy(rows_v, out_hbm.at[pl.ds(base, b_per_w)])

    return sc_gather


_sc_gather = _make_sc_gather()


def _prep_kernel(scores_ref, unc_ref, boxes_ref, wcol_ref, negw_ref, table_ref):
    u = unc_ref[...]
    m = (u[:, 0:1] + u[:, 1:2] + u[:, 2:3] + u[:, 3:4]) * 0.25
    w = scores_ref[...].T * (1.0 - jax.nn.sigmoid(m))
    wcol_ref[...] = w
    negw_ref[0:1, 0:_N] = -w.T
    negw_ref[0:1, _N:_NPAD] = jnp.full((1, _NPAD - _N), jnp.inf, jnp.float32)
    b = boxes_ref[...]
    a3 = (b[:, 2:3] - b[:, 0:1]) * (b[:, 3:4] - b[:, 1:2]) / 3.0
    table_ref[...] = jnp.zeros((_NPAD, _D), jnp.float32)
    table_ref[pl.ds(0, _N), 0:4] = b
    table_ref[pl.ds(0, _N), 4:5] = a3


def _nms_kernel(st_ref, keep_ref, bt_ref, tile_ref, supp_ref):
    tri = (
        jax.lax.broadcasted_iota(jnp.int32, (_B, _B), 1)
        > jax.lax.broadcasted_iota(jnp.int32, (_B, _B), 0)
    )
    supp_ref[...] = jnp.zeros((1, _NPAD), jnp.float32)

    def bt_body(j, carry):
        c0 = j * _B
        bt_ref[:, pl.ds(c0, _B)] = st_ref[pl.ds(c0, _B), :].T
        return carry

    jax.lax.fori_loop(0, _NB, bt_body, 0)

    def block_body(bi, carry):
        r0 = bi * _B
        rx0 = st_ref[pl.ds(r0, _B), 0:1]
        ry0 = st_ref[pl.ds(r0, _B), 1:2]
        rx1 = st_ref[pl.ds(r0, _B), 2:3]
        ry1 = st_ref[pl.ds(r0, _B), 3:4]
        ra3 = st_ref[pl.ds(r0, _B), 4:5]

        def iou_bin(c0):
            cx0 = bt_ref[0:1, pl.ds(c0, _B)]
            cy0 = bt_ref[1:2, pl.ds(c0, _B)]
            cx1 = bt_ref[2:3, pl.ds(c0, _B)]
            cy1 = bt_ref[3:4, pl.ds(c0, _B)]
            ca3 = bt_ref[4:5, pl.ds(c0, _B)]
            w = jnp.maximum(jnp.minimum(rx1, cx1) - jnp.maximum(rx0, cx0), 0.0)
            h = jnp.maximum(jnp.minimum(ry1, cy1) - jnp.maximum(ry0, cy0), 0.0)
            return w * h > (ra3 + ca3)

        tile_ref[...] = jnp.where(
            iou_bin(r0) & tri, 1.0, 0.0
        ).astype(jnp.bfloat16)
        active = 1.0 - supp_ref[:, pl.ds(r0, _B)]

        def w_cond(c):
            return c[1]

        def w_body(c):
            k, _ = c
            t = jnp.dot(
                k.astype(jnp.bfloat16),
                tile_ref[...],
                preferred_element_type=jnp.float32,
            )
            k_new = active * jnp.where(t < 0.5, 1.0, 0.0)
            return k_new, jnp.any(k_new != k)

        k, _ = jax.lax.while_loop(w_cond, w_body, (active, jnp.bool_(True)))
        supp_ref[:, pl.ds(r0, _B)] = 1.0 - k
        kb = k.astype(jnp.bfloat16)

        def col_body(j, carry2):
            @pl.when(j > bi)
            def _():
                c0 = j * _B
                tile = jnp.where(iou_bin(c0), 1.0, 0.0).astype(jnp.bfloat16)
                t = jnp.dot(kb, tile, preferred_element_type=jnp.float32)
                supp_ref[:, pl.ds(c0, _B)] = jnp.maximum(
                    supp_ref[:, pl.ds(c0, _B)],
                    jnp.where(t > 0.5, 1.0, 0.0),
                )

            return carry2

        jax.lax.fori_loop(0, _NB, col_body, carry, unroll=5)
        return carry

    jax.lax.fori_loop(0, _NB, block_body, 0)
    keep_ref[...] = 1.0 - supp_ref[...]


def _assemble_kernel(boxes_ref, scores_ref, unc_ref, w_ref, cont_ref, keep_ref, out_ref):
    k = keep_ref[0:1, 0:_N].T
    out_ref[:, 0:4] = boxes_ref[...] * k
    out_ref[:, 4:5] = scores_ref[...].T * k
    out_ref[:, 5:9] = unc_ref[...] * k
    out_ref[:, 9:10] = w_ref[...] * k
    out_ref[:, 10:74] = cont_ref[...] * k


def kernel(contours, scores, boxes, uncertainties):
    wcol, negw, table = pl.pallas_call(
        _prep_kernel,
        out_shape=(
            jax.ShapeDtypeStruct((_N, 1), jnp.float32),
            jax.ShapeDtypeStruct((1, _NPAD), jnp.float32),
            jax.ShapeDtypeStruct((_NPAD, _D), jnp.float32),
        ),
    )(scores.reshape(1, _N), uncertainties, boxes)
    order_pad = jnp.argsort(negw.reshape(_NPAD)).astype(jnp.int32)
    sorted_table = _sc_gather(table, order_pad)
    keep_row = pl.pallas_call(
        _nms_kernel,
        out_shape=jax.ShapeDtypeStruct((1, _NPAD), jnp.float32),
        scratch_shapes=[
            pltpu.VMEM((_D, _NPAD), jnp.float32),
            pltpu.VMEM((_B, _B), jnp.bfloat16),
            pltpu.VMEM((1, _NPAD), jnp.float32),
        ],
    )(sorted_table)
    _, keep_pad = jax.lax.sort((order_pad, keep_row.reshape(_NPAD)), num_keys=1)
    out = pl.pallas_call(
        _assemble_kernel,
        out_shape=jax.ShapeDtypeStruct((_N, 74), jnp.float32),
    )(
        boxes,
        scores.reshape(1, _N),
        uncertainties,
        wcol,
        contours.reshape(_N, -1),
        keep_pad.reshape(1, _NPAD),
    )
    return out
